# Initial kernel scaffold; baseline (speedup 1.0000x reference)
import functools
import math

import jax
import jax.numpy as jnp
from jax import lax
from jax.experimental import pallas as pl
from jax.experimental.pallas import tpu as pltpu
from jax.experimental.pallas import tpu_sc as plsc

_SPA_EMBED_DIM = 32
_FREQ = 16
_EPS = 5
_MAX_RADIUS = 10000
_NUM_ROWS = (_FREQ + _EPS) ** 2
_BATCH = 16384
_NUM_PT = 50
_B = _BATCH * _NUM_PT

_LOG_DIST_INTERVAL = math.log(float(_MAX_RADIUS + 1) - math.log(1.0)) / (_FREQ * 1.0)
_ANGLE_INTERVAL = math.pi * 2.0 / (_FREQ * 1.0)

_IDX_ROWS = 6400
_IDX_BLK = 800

_NC, _NS = 2, 16
_NW = _NC * _NS
_BPW = _B // _NW
_CHUNK = 1024
_NCHUNK = _BPW // _CHUNK


def _idx_body(x_ref, y_ref, idx_ref):
    x = x_ref[...]
    y = y_ref[...]
    dist = jnp.log(jnp.sqrt(x * x + y * y) + 1.0)
    row = jnp.floor(dist / _LOG_DIST_INTERVAL)
    angle = jnp.arctan2(y, x) + math.pi
    col = jnp.floor(angle / _ANGLE_INTERVAL)
    idx = (row * _FREQ + col).astype(jnp.int32)
    idx_ref[...] = jnp.clip(idx, 0, _NUM_ROWS - 1)


def _compute_idx(x2d, y2d):
    return pl.pallas_call(
        _idx_body,
        grid=(_IDX_ROWS // _IDX_BLK,),
        in_specs=[
            pl.BlockSpec((_IDX_BLK, 128), lambda i: (i, 0)),
            pl.BlockSpec((_IDX_BLK, 128), lambda i: (i, 0)),
        ],
        out_specs=pl.BlockSpec((_IDX_BLK, 128), lambda i: (i, 0)),
        out_shape=jax.ShapeDtypeStruct((_IDX_ROWS, 128), jnp.int32),
    )(x2d, y2d)


def _gather_body(table_hbm, idx_hbm, out_hbm, idx_v, rows_v, sem):
    wid = lax.axis_index("s") * _NC + lax.axis_index("c")
    base = wid * _BPW

    def step(i, carry):
        off = base + i * _CHUNK
        pltpu.sync_copy(idx_hbm.at[pl.ds(off, _CHUNK)], idx_v)
        pltpu.async_copy(table_hbm.at[idx_v], rows_v, sem).wait()
        pltpu.sync_copy(rows_v, out_hbm.at[pl.ds(off, _CHUNK)])
        return carry

    lax.fori_loop(0, _NCHUNK, step, 0)


_sc_gather = functools.partial(
    pl.kernel,
    mesh=plsc.VectorSubcoreMesh(core_axis_name="c", subcore_axis_name="s"),
    out_type=jax.ShapeDtypeStruct((_B, _SPA_EMBED_DIM), jnp.float32),
    scratch_types=[
        pltpu.VMEM((_CHUNK,), jnp.int32),
        pltpu.VMEM((_CHUNK, _SPA_EMBED_DIM), jnp.float32),
        pltpu.SemaphoreType.DMA,
    ],
)(_gather_body)


def kernel(coords, table):
    x2d = coords[:, :, 0].reshape(_IDX_ROWS, 128)
    y2d = coords[:, :, 1].reshape(_IDX_ROWS, 128)
    idx = _compute_idx(x2d, y2d).reshape(_B)
    rows = _sc_gather(table, idx)
    return rows.reshape(_BATCH, _NUM_PT, _SPA_EMBED_DIM)


# speedup vs baseline: 1.4326x; 1.4326x over previous
import functools
import math

import jax
import jax.numpy as jnp
from jax import lax
from jax.experimental import pallas as pl
from jax.experimental.pallas import tpu as pltpu
from jax.experimental.pallas import tpu_sc as plsc

_SPA_EMBED_DIM = 32
_FREQ = 16
_EPS = 5
_MAX_RADIUS = 10000
_NUM_ROWS = (_FREQ + _EPS) ** 2
_BATCH = 16384
_NUM_PT = 50
_B = _BATCH * _NUM_PT

_LOG_DIST_INTERVAL = math.log(float(_MAX_RADIUS + 1) - math.log(1.0)) / (_FREQ * 1.0)
_ANGLE_INTERVAL = math.pi * 2.0 / (_FREQ * 1.0)

_IDX_ROWS = 6400
_IDX_BLK = 800

_NC, _NS = 2, 16
_NW = _NC * _NS
_BPW = _B // _NW
_CHUNK = 1024
_NCHUNK = _BPW // _CHUNK


def _idx_body(x_ref, y_ref, idx_ref):
    x = x_ref[...]
    y = y_ref[...]
    dist = jnp.log(jnp.sqrt(x * x + y * y) + 1.0)
    row = jnp.floor(dist / _LOG_DIST_INTERVAL)
    angle = jnp.arctan2(y, x) + math.pi
    col = jnp.floor(angle / _ANGLE_INTERVAL)
    idx = (row * _FREQ + col).astype(jnp.int32)
    idx_ref[...] = jnp.clip(idx, 0, _NUM_ROWS - 1)


def _compute_idx(x2d, y2d):
    return pl.pallas_call(
        _idx_body,
        grid=(_IDX_ROWS // _IDX_BLK,),
        in_specs=[
            pl.BlockSpec((_IDX_BLK, 128), lambda i: (i, 0)),
            pl.BlockSpec((_IDX_BLK, 128), lambda i: (i, 0)),
        ],
        out_specs=pl.BlockSpec((_IDX_BLK, 128), lambda i: (i, 0)),
        out_shape=jax.ShapeDtypeStruct((_IDX_ROWS, 128), jnp.int32),
    )(x2d, y2d)


def _gather_body(table_hbm, idx_hbm, out_hbm, idx_v, rows_v, sem):
    wid = lax.axis_index("s") * _NC + lax.axis_index("c")
    base = wid * _BPW

    def step(i, carry):
        off = base + i * _CHUNK
        pltpu.sync_copy(idx_hbm.at[pl.ds(off, _CHUNK)], idx_v)
        pltpu.async_copy(table_hbm.at[idx_v], rows_v, sem).wait()
        pltpu.sync_copy(rows_v, out_hbm.at[pl.ds(off, _CHUNK)])
        return carry

    lax.fori_loop(0, _NCHUNK, step, 0)


@functools.cache
def _sc_gather():
    return pl.kernel(
        _gather_body,
        mesh=plsc.VectorSubcoreMesh(core_axis_name="c", subcore_axis_name="s"),
        out_type=jax.ShapeDtypeStruct((_B, _SPA_EMBED_DIM), jnp.float32),
        scratch_types=[
            pltpu.VMEM((_CHUNK,), jnp.int32),
            pltpu.VMEM((_CHUNK, _SPA_EMBED_DIM), jnp.float32),
            pltpu.SemaphoreType.DMA,
        ],
        compiler_params=pltpu.CompilerParams(use_tc_tiling_on_sc=False),
    )


def kernel(coords, table):
    x2d = coords[:, :, 0].reshape(_IDX_ROWS, 128)
    y2d = coords[:, :, 1].reshape(_IDX_ROWS, 128)
    idx = _compute_idx(x2d, y2d).reshape(_B)
    rows = _sc_gather()(table, idx)
    return rows.reshape(_BATCH, _NUM_PT, _SPA_EMBED_DIM)


# speedup vs baseline: 1.4355x; 1.0021x over previous
import functools
import math

import jax
import jax.numpy as jnp
from jax import lax
from jax.experimental import pallas as pl
from jax.experimental.pallas import tpu as pltpu
from jax.experimental.pallas import tpu_sc as plsc

_SPA_EMBED_DIM = 32
_FREQ = 16
_EPS = 5
_MAX_RADIUS = 10000
_NUM_ROWS = (_FREQ + _EPS) ** 2
_BATCH = 16384
_NUM_PT = 50
_B = _BATCH * _NUM_PT

_LOG_DIST_INTERVAL = math.log(float(_MAX_RADIUS + 1) - math.log(1.0)) / (_FREQ * 1.0)
_ANGLE_INTERVAL = math.pi * 2.0 / (_FREQ * 1.0)

_IDX_ROWS = 6400
_IDX_BLK = 800

_NC, _NS = 2, 16
_NW = _NC * _NS
_BPW = _B // _NW
_CHUNK = 1280
_NCHUNK = _BPW // _CHUNK


def _idx_body(x_ref, y_ref, idx_ref):
    x = x_ref[...]
    y = y_ref[...]
    dist = jnp.log(jnp.sqrt(x * x + y * y) + 1.0)
    row = jnp.floor(dist / _LOG_DIST_INTERVAL)
    angle = jnp.arctan2(y, x) + math.pi
    col = jnp.floor(angle / _ANGLE_INTERVAL)
    idx = (row * _FREQ + col).astype(jnp.int32)
    idx_ref[...] = jnp.clip(idx, 0, _NUM_ROWS - 1)


def _compute_idx(x2d, y2d):
    return pl.pallas_call(
        _idx_body,
        grid=(_IDX_ROWS // _IDX_BLK,),
        in_specs=[
            pl.BlockSpec((_IDX_BLK, 128), lambda i: (i, 0)),
            pl.BlockSpec((_IDX_BLK, 128), lambda i: (i, 0)),
        ],
        out_specs=pl.BlockSpec((_IDX_BLK, 128), lambda i: (i, 0)),
        out_shape=jax.ShapeDtypeStruct((_IDX_ROWS, 128), jnp.int32),
    )(x2d, y2d)


def _gather_body(table_hbm, idx_hbm, out_hbm, idx_v, rows0, rows1, gs0, gs1, os0, os1):
    wid = lax.axis_index("s") * _NC + lax.axis_index("c")
    base = wid * _BPW
    rows = (rows0, rows1)
    gs = (gs0, gs1)
    os = (os0, os1)

    pltpu.sync_copy(idx_hbm.at[pl.ds(base, _BPW)], idx_v)

    def g_desc(i, b):
        return pltpu.make_async_copy(
            table_hbm.at[idx_v.at[pl.ds(i * _CHUNK, _CHUNK)]], rows[b], gs[b]
        )

    def o_desc(i, b):
        return pltpu.make_async_copy(
            rows[b], out_hbm.at[pl.ds(base + i * _CHUNK, _CHUNK)], os[b]
        )

    g_desc(0, 0).start()
    for i in range(_NCHUNK):
        b = i & 1
        g_desc(i, b).wait()
        if i + 1 < _NCHUNK:
            if i >= 1:
                o_desc(i - 1, 1 - b).wait()
            g_desc(i + 1, 1 - b).start()
        o_desc(i, b).start()
    o_desc(_NCHUNK - 2, (_NCHUNK - 2) & 1).wait()
    o_desc(_NCHUNK - 1, (_NCHUNK - 1) & 1).wait()


@functools.cache
def _sc_gather():
    return pl.kernel(
        _gather_body,
        mesh=plsc.VectorSubcoreMesh(core_axis_name="c", subcore_axis_name="s"),
        out_type=jax.ShapeDtypeStruct((_B, _SPA_EMBED_DIM), jnp.float32),
        scratch_types=[
            pltpu.VMEM((_BPW,), jnp.int32),
            pltpu.VMEM((_CHUNK, _SPA_EMBED_DIM), jnp.float32),
            pltpu.VMEM((_CHUNK, _SPA_EMBED_DIM), jnp.float32),
            pltpu.SemaphoreType.DMA,
            pltpu.SemaphoreType.DMA,
            pltpu.SemaphoreType.DMA,
            pltpu.SemaphoreType.DMA,
        ],
        compiler_params=pltpu.CompilerParams(use_tc_tiling_on_sc=False),
    )


def kernel(coords, table):
    x2d = coords[:, :, 0].reshape(_IDX_ROWS, 128)
    y2d = coords[:, :, 1].reshape(_IDX_ROWS, 128)
    idx = _compute_idx(x2d, y2d).reshape(_B)
    rows = _sc_gather()(table, idx)
    return rows.reshape(_BATCH, _NUM_PT, _SPA_EMBED_DIM)


# speedup vs baseline: 2.5305x; 1.7628x over previous
import functools
import math

import jax
import jax.numpy as jnp
from jax import lax
from jax.experimental import pallas as pl
from jax.experimental.pallas import tpu as pltpu
from jax.experimental.pallas import tpu_sc as plsc

_SPA_EMBED_DIM = 32
_FREQ = 16
_EPS = 5
_MAX_RADIUS = 10000
_NUM_ROWS = (_FREQ + _EPS) ** 2
_BATCH = 16384
_NUM_PT = 50
_B = _BATCH * _NUM_PT

_LOG_DIST_INTERVAL = math.log(float(_MAX_RADIUS + 1) - math.log(1.0)) / (_FREQ * 1.0)
_ANGLE_INTERVAL = math.pi * 2.0 / (_FREQ * 1.0)

_IDX_ROWS = 6400
_IDX_BLK = 800

_NC, _NS = 2, 16
_NW = _NC * _NS
_BPW = _B // _NW
_CHUNK = 1280
_NCHUNK = _BPW // _CHUNK


def _idx_body(x_ref, y_ref, idx_ref):
    x = x_ref[...]
    y = y_ref[...]
    dist = jnp.log(jnp.sqrt(x * x + y * y) + 1.0)
    row = jnp.floor(dist / _LOG_DIST_INTERVAL)
    angle = jnp.arctan2(y, x) + math.pi
    col = jnp.floor(angle / _ANGLE_INTERVAL)
    idx = (row * _FREQ + col).astype(jnp.int32)
    idx_ref[...] = jnp.clip(idx, 0, _NUM_ROWS - 1)


def _compute_idx(x2d, y2d):
    return pl.pallas_call(
        _idx_body,
        grid=(_IDX_ROWS // _IDX_BLK,),
        in_specs=[
            pl.BlockSpec((_IDX_BLK, 128), lambda i: (i, 0)),
            pl.BlockSpec((_IDX_BLK, 128), lambda i: (i, 0)),
        ],
        out_specs=pl.BlockSpec((_IDX_BLK, 128), lambda i: (i, 0)),
        out_shape=jax.ShapeDtypeStruct((_IDX_ROWS, 128), jnp.int32),
    )(x2d, y2d)


_GROUPS = _CHUNK // 16


def _gather_body(table_hbm, idx_hbm, out_hbm, table_v, idx_v, rows0, rows1, os0, os1):
    wid = lax.axis_index("s") * _NC + lax.axis_index("c")
    base = wid * _BPW
    rows = (rows0, rows1)
    os = (os0, os1)

    pltpu.sync_copy(table_hbm, table_v)
    pltpu.sync_copy(idx_hbm.at[pl.ds(base, _BPW)], idx_v)

    lane = lax.iota(jnp.int32, 16)
    lane32 = lane * _SPA_EMBED_DIM

    def o_desc(i, b):
        return pltpu.make_async_copy(
            rows[b],
            out_hbm.at[pl.ds((base + i * _CHUNK) * _SPA_EMBED_DIM, _CHUNK * _SPA_EMBED_DIM)],
            os[b],
        )

    def fill_chunk(i, b):
        cbase = i * _CHUNK

        def group(g, carry):
            v = idx_v[pl.ds(cbase + g * 16, 16)]
            offs = lane32 + g * (16 * _SPA_EMBED_DIM)
            for j in range(_SPA_EMBED_DIM):
                jv = jnp.full((16,), j, jnp.int32)
                vals = plsc.load_gather(table_v, [v, jv])
                plsc.store_scatter(rows[b], [offs + j], vals)
            return carry

        lax.fori_loop(0, _GROUPS, group, 0)

    for i in range(_NCHUNK):
        b = i & 1
        if i >= 2:
            o_desc(i - 2, b).wait()
        fill_chunk(i, b)
        o_desc(i, b).start()
    o_desc(_NCHUNK - 2, (_NCHUNK - 2) & 1).wait()
    o_desc(_NCHUNK - 1, (_NCHUNK - 1) & 1).wait()


@functools.cache
def _sc_gather():
    return pl.kernel(
        _gather_body,
        mesh=plsc.VectorSubcoreMesh(core_axis_name="c", subcore_axis_name="s"),
        out_type=jax.ShapeDtypeStruct((_B * _SPA_EMBED_DIM,), jnp.float32),
        scratch_types=[
            pltpu.VMEM((_NUM_ROWS, _SPA_EMBED_DIM), jnp.float32),
            pltpu.VMEM((_BPW,), jnp.int32),
            pltpu.VMEM((_CHUNK * _SPA_EMBED_DIM,), jnp.float32),
            pltpu.VMEM((_CHUNK * _SPA_EMBED_DIM,), jnp.float32),
            pltpu.SemaphoreType.DMA,
            pltpu.SemaphoreType.DMA,
        ],
        compiler_params=pltpu.CompilerParams(
            use_tc_tiling_on_sc=False, needs_layout_passes=False
        ),
    )


def kernel(coords, table):
    x2d = coords[:, :, 0].reshape(_IDX_ROWS, 128)
    y2d = coords[:, :, 1].reshape(_IDX_ROWS, 128)
    idx = _compute_idx(x2d, y2d).reshape(_B)
    flat = _sc_gather()(table, idx)
    return flat.reshape(_BATCH, _NUM_PT, _SPA_EMBED_DIM)
